# Initial kernel scaffold; baseline (speedup 1.0000x reference)
import jax
import jax.numpy as jnp
from jax.experimental import pallas as pl


def kernel(pattern_ids, pattern_metadata, emb_table, W1, b1, W2, b2, gamma, beta):
    raise NotImplementedError("write your pallas kernel here")


# speedup vs baseline: 2.2852x; 2.2852x over previous
import functools

import jax
import jax.numpy as jnp
from jax.experimental import pallas as pl
from jax.experimental.pallas import tpu as pltpu
from jax.experimental.pallas import tpu_sc as plsc

_EMBED = 64
_NMETA = 16
_CTX = 128
_P = 50

_GATHER_WINDOW = 128
_TC_ROWS = 800


def _sc_gather(emb_table, flat_ids):
    n = flat_ids.shape[1]
    mesh = plsc.VectorSubcoreMesh(core_axis_name="c", subcore_axis_name="s")

    @functools.partial(
        pl.kernel,
        out_type=jax.ShapeDtypeStruct((n, _EMBED), emb_table.dtype),
        mesh=mesh,
        compiler_params=pltpu.CompilerParams(use_tc_tiling_on_sc=False),
    )
    def gather_kernel(tbl_hbm, idx_hbm, out_hbm):
        def body(i_vmem, o_vmem):
            pltpu.sync_copy(tbl_hbm.at[i_vmem.at[0]], o_vmem)

        pltpu.emit_pipeline(
            body,
            grid=(n // _GATHER_WINDOW,),
            in_specs=[
                pl.BlockSpec((1, _GATHER_WINDOW), index_map=lambda i: (0, i))
            ],
            out_specs=[
                pl.BlockSpec((_GATHER_WINDOW, _EMBED), index_map=lambda i: (i, 0))
            ],
            core_axis_name=("c", "s"),
            dimension_semantics=(pltpu.PARALLEL,),
        )(idx_hbm, out_hbm)

    return gather_kernel(emb_table, flat_ids)


def _tc_mlp_body(emb_ref, meta_ref, w1a_ref, w1b_ref, b1_ref, w2_ref, b2_ref,
                 gamma_ref, beta_ref, out_ref):
    h = (
        jnp.dot(emb_ref[...], w1a_ref[...], preferred_element_type=jnp.float32)
        + jnp.dot(meta_ref[...], w1b_ref[...], preferred_element_type=jnp.float32)
        + b1_ref[...]
    )
    h = jnp.maximum(h, 0.0)
    h = jnp.dot(h, w2_ref[...], preferred_element_type=jnp.float32) + b2_ref[...]
    mean = jnp.mean(h, axis=-1, keepdims=True)
    d = h - mean
    var = jnp.mean(d * d, axis=-1, keepdims=True)
    y = d * jax.lax.rsqrt(var + 1e-5) * gamma_ref[...] + beta_ref[...]
    for g in range(_TC_ROWS // _P):
        out_ref[g, :] = jnp.max(y[g * _P:(g + 1) * _P, :], axis=0)


def _tc_mlp(embeds, meta2d, w1a, w1b, b1, w2, b2, gamma, beta):
    n = embeds.shape[0]
    groups = _TC_ROWS // _P
    fixed = lambda i: (0, 0)
    return pl.pallas_call(
        _tc_mlp_body,
        grid=(n // _TC_ROWS,),
        in_specs=[
            pl.BlockSpec((_TC_ROWS, _EMBED), lambda i: (i, 0)),
            pl.BlockSpec((_TC_ROWS, _NMETA), lambda i: (i, 0)),
            pl.BlockSpec((_EMBED, _CTX), fixed),
            pl.BlockSpec((_NMETA, _CTX), fixed),
            pl.BlockSpec((1, _CTX), fixed),
            pl.BlockSpec((_CTX, _CTX), fixed),
            pl.BlockSpec((1, _CTX), fixed),
            pl.BlockSpec((1, _CTX), fixed),
            pl.BlockSpec((1, _CTX), fixed),
        ],
        out_specs=pl.BlockSpec((groups, _CTX), lambda i: (i, 0)),
        out_shape=jax.ShapeDtypeStruct((n // _P, _CTX), jnp.float32),
    )(embeds, meta2d, w1a, w1b, b1, w2, b2, gamma, beta)


def kernel(pattern_ids, pattern_metadata, emb_table, W1, b1, W2, b2, gamma, beta):
    bsz, p = pattern_ids.shape
    n = bsz * p
    flat_ids = pattern_ids.reshape(1, n).astype(jnp.int32)
    embeds = _sc_gather(emb_table, flat_ids)
    meta2d = pattern_metadata.reshape(n, _NMETA)
    w1a = W1[:_EMBED]
    w1b = W1[_EMBED:]
    out = _tc_mlp(
        embeds, meta2d, w1a, w1b,
        b1.reshape(1, _CTX), W2, b2.reshape(1, _CTX),
        gamma.reshape(1, _CTX), beta.reshape(1, _CTX),
    )
    return out


# speedup vs baseline: 3.7928x; 1.6597x over previous
import functools

import jax
import jax.numpy as jnp
from jax.experimental import pallas as pl
from jax.experimental.pallas import tpu as pltpu
from jax.experimental.pallas import tpu_sc as plsc

_EMBED = 64
_NMETA = 16
_CTX = 128
_P = 50

_GATHER_WINDOW = 128
_BBLK = 256


def _sc_gather(emb_table, flat_ids):
    n = flat_ids.shape[1]
    mesh = plsc.VectorSubcoreMesh(core_axis_name="c", subcore_axis_name="s")

    @functools.partial(
        pl.kernel,
        out_type=jax.ShapeDtypeStruct((n, _EMBED), emb_table.dtype),
        mesh=mesh,
        compiler_params=pltpu.CompilerParams(use_tc_tiling_on_sc=False),
    )
    def gather_kernel(tbl_hbm, idx_hbm, out_hbm):
        def body(i_vmem, o_vmem):
            pltpu.sync_copy(tbl_hbm.at[i_vmem.at[0]], o_vmem)

        pltpu.emit_pipeline(
            body,
            grid=(n // _GATHER_WINDOW,),
            in_specs=[
                pl.BlockSpec((1, _GATHER_WINDOW), index_map=lambda i: (0, i))
            ],
            out_specs=[
                pl.BlockSpec((_GATHER_WINDOW, _EMBED), index_map=lambda i: (i, 0))
            ],
            core_axis_name=("c", "s"),
            dimension_semantics=(pltpu.PARALLEL,),
        )(idx_hbm, out_hbm)

    return gather_kernel(emb_table, flat_ids)


def _tc_mlp_body(emb_ref, meta_ref, w1a_ref, w1b_ref, b1_ref, w2_ref, b2_ref,
                 gamma_ref, beta_ref, out_ref):
    p, nb = emb_ref.shape[0], emb_ref.shape[1]
    emb = emb_ref[...].reshape(p * nb, _EMBED)
    meta = meta_ref[...].reshape(p * nb, _NMETA)
    h = (
        jnp.dot(emb, w1a_ref[...], preferred_element_type=jnp.float32)
        + jnp.dot(meta, w1b_ref[...], preferred_element_type=jnp.float32)
        + b1_ref[...]
    )
    h = jnp.maximum(h, 0.0)
    h = jnp.dot(h, w2_ref[...], preferred_element_type=jnp.float32) + b2_ref[...]
    mean = jnp.mean(h, axis=-1, keepdims=True)
    d = h - mean
    var = jnp.mean(d * d, axis=-1, keepdims=True)
    y = d * jax.lax.rsqrt(var + 1e-5) * gamma_ref[...] + beta_ref[...]
    out_ref[...] = jnp.max(y.reshape(p, nb, _CTX), axis=0)


def _tc_mlp(embeds3, meta3, w1a, w1b, b1, w2, b2, gamma, beta):
    p, bsz = embeds3.shape[0], embeds3.shape[1]
    fixed = lambda i: (0, 0)
    return pl.pallas_call(
        _tc_mlp_body,
        grid=(bsz // _BBLK,),
        in_specs=[
            pl.BlockSpec((p, _BBLK, _EMBED), lambda i: (0, i, 0)),
            pl.BlockSpec((p, _BBLK, _NMETA), lambda i: (0, i, 0)),
            pl.BlockSpec((_EMBED, _CTX), fixed),
            pl.BlockSpec((_NMETA, _CTX), fixed),
            pl.BlockSpec((1, _CTX), fixed),
            pl.BlockSpec((_CTX, _CTX), fixed),
            pl.BlockSpec((1, _CTX), fixed),
            pl.BlockSpec((1, _CTX), fixed),
            pl.BlockSpec((1, _CTX), fixed),
        ],
        out_specs=pl.BlockSpec((_BBLK, _CTX), lambda i: (i, 0)),
        out_shape=jax.ShapeDtypeStruct((bsz, _CTX), jnp.float32),
    )(embeds3, meta3, w1a, w1b, b1, w2, b2, gamma, beta)


def kernel(pattern_ids, pattern_metadata, emb_table, W1, b1, W2, b2, gamma, beta):
    bsz, p = pattern_ids.shape
    n = bsz * p
    flat_ids = pattern_ids.T.reshape(1, n).astype(jnp.int32)
    embeds = _sc_gather(emb_table, flat_ids)
    embeds3 = embeds.reshape(p, bsz, _EMBED)
    meta3 = pattern_metadata.transpose(1, 0, 2)
    w1a = W1[:_EMBED]
    w1b = W1[_EMBED:]
    out = _tc_mlp(
        embeds3, meta3, w1a, w1b,
        b1.reshape(1, _CTX), W2, b2.reshape(1, _CTX),
        gamma.reshape(1, _CTX), beta.reshape(1, _CTX),
    )
    return out
